# baseline (speedup 1.0000x reference)
import functools

import jax
import jax.numpy as jnp
import numpy as np
from jax import lax
from jax.experimental import pallas as pl
from jax.experimental.pallas import tpu as pltpu
from jax.experimental.pallas import tpu_sc as plsc

_N = 16
_A = 3
_H = 64
_W = 64
_NCLS = 80
_S = _H * _W
_NB = _A * _S
_K = 300
_PAD = 384
_MAXDET = 100
_CONF = 0.5
_NMS_T = 0.4
_IMG = 512.0
_AW = (10.0, 16.0, 33.0)
_AH = (13.0, 30.0, 23.0)
_ONE_BITS = np.int32(np.float32(1.0).view(np.int32))


def _extract_body(x_ref, o_ref):
    xb = x_ref[0]
    zero_row = jnp.zeros((_S,), jnp.float32)
    for a in range(_A):
        base = a * (_NCLS + 5)
        cls = xb[:, base + 5:base + 5 + _NCLS]
        cmax = jnp.max(cls, axis=1)
        li = lax.broadcasted_iota(jnp.int32, (_S, _NCLS), 1)
        cid = jnp.min(jnp.where(cls == cmax[:, None], li, _NCLS), axis=1)

        def plane(k):
            return jnp.max(xb[:, base + k:base + k + 1], axis=1)

        r = a * 8
        o_ref[0, r + 0] = plane(0)
        o_ref[0, r + 1] = plane(1)
        o_ref[0, r + 2] = plane(2)
        o_ref[0, r + 3] = plane(3)
        o_ref[0, r + 4] = plane(4)
        o_ref[0, r + 5] = cmax
        o_ref[0, r + 6] = cid.astype(jnp.float32)
        o_ref[0, r + 7] = zero_row


def _extract(xt):
    return pl.pallas_call(
        _extract_body,
        grid=(_N,),
        in_specs=[pl.BlockSpec((1, _S, _A * (_NCLS + 5)), lambda n: (n, 0, 0))],
        out_specs=pl.BlockSpec((1, 24, _S), lambda n: (n, 0, 0)),
        out_shape=jax.ShapeDtypeStruct((_N, 24, _S), jnp.float32),
    )(xt)


def _select_body(t_ref, s_ref, x1_ref, y1_ref, x2_ref, y2_ref, c_ref, d_ref):
    t = t_ref[...]

    def field(j):
        return jnp.concatenate([t[:, j, :], t[:, 8 + j, :], t[:, 16 + j, :]],
                               axis=1)

    tx = field(0)
    ty = field(1)
    tw = field(2)
    th = field(3)
    tobj = field(4)
    cmax = field(5)
    cid = field(6)

    lin = lax.broadcasted_iota(jnp.int32, (_N, _NB), 1)
    gx = (lin % _W).astype(jnp.float32)
    gy = ((lin % _S) // _W).astype(jnp.float32)
    aw = jnp.where(lin < _S, _AW[0], jnp.where(lin < 2 * _S, _AW[1], _AW[2]))
    ah = jnp.where(lin < _S, _AH[0], jnp.where(lin < 2 * _S, _AH[1], _AH[2]))

    px = jax.nn.sigmoid(tx) + gx
    py = jax.nn.sigmoid(ty) + gy
    pw = jnp.exp(tw) * aw
    ph = jnp.exp(th) * ah
    bx = (px * _IMG) * _IMG
    by = (py * _IMG) * _IMG
    bw = (pw * _IMG) * _IMG
    bh = (ph * _IMG) * _IMG
    score = jax.nn.sigmoid(tobj) * jax.nn.sigmoid(cmax)

    s_ref[...] = score
    x1_ref[...] = bx - bw / 2.0
    y1_ref[...] = by - bh / 2.0
    x2_ref[...] = bx + bw / 2.0
    y2_ref[...] = by + bh / 2.0
    c_ref[...] = cid

    bits = lax.bitcast_convert_type(score, jnp.int32)

    def b1(_, carry):
        lo, hi = carry
        mid = (lo + hi) // 2
        cnt = jnp.sum((bits >= mid).astype(jnp.int32), axis=1, keepdims=True)
        ok = cnt >= _K
        return jnp.where(ok, mid, lo), jnp.where(ok, hi, mid)

    lo0 = jnp.zeros((_N, 1), jnp.int32)
    hi0 = jnp.full((_N, 1), _ONE_BITS + np.int32(1), jnp.int32)
    tbits, _ = lax.fori_loop(0, 31, b1, (lo0, hi0))
    cnt_gt = jnp.sum((bits >= tbits + 1).astype(jnp.int32), axis=1,
                     keepdims=True)
    need = _K - cnt_gt
    at_t = bits == tbits

    def b2(_, carry):
        lo2, hi2 = carry
        mid = (lo2 + hi2) // 2
        cnt = jnp.sum((at_t & (lin <= mid)).astype(jnp.int32), axis=1,
                      keepdims=True)
        ok = cnt >= need
        return jnp.where(ok, lo2, mid), jnp.where(ok, mid, hi2)

    lo20 = jnp.full((_N, 1), -1, jnp.int32)
    hi20 = jnp.full((_N, 1), _NB - 1, jnp.int32)
    _, cstar = lax.fori_loop(0, 15, b2, (lo20, hi20))
    elig = (bits > tbits) | (at_t & (lin <= cstar))

    e_i = elig.astype(jnp.int32)
    c = e_i
    k = 1
    while k < _NB:
        c = c + jnp.concatenate(
            [jnp.zeros((_N, k), jnp.int32), c[:, :-k]], axis=1)
        k *= 2
    pexcl = c - e_i
    d_ref[...] = jnp.where(elig, pexcl, (_PAD - 16) + (lin & 15))


def _select(t):
    f = jax.ShapeDtypeStruct((_N, _NB), jnp.float32)
    fi = jax.ShapeDtypeStruct((_N, _NB), jnp.int32)
    return pl.pallas_call(
        _select_body,
        out_shape=[f] * 6 + [fi],
    )(t)


def _compact_body(s_h, x1_h, y1_h, x2_h, y2_h, c_h, d_h,
                  os_h, ox1_h, oy1_h, ox2_h, oy2_h, oc_h,
                  s_v, x1_v, y1_v, x2_v, y2_v, c_v, d_v,
                  bs, bx1, by1, bx2, by2, bc):
    wid = lax.axis_index("s") * 2 + lax.axis_index("c")

    @pl.when(wid < _N)
    def _():
        img = wid
        pltpu.sync_copy(s_h.at[img], s_v)
        pltpu.sync_copy(x1_h.at[img], x1_v)
        pltpu.sync_copy(y1_h.at[img], y1_v)
        pltpu.sync_copy(x2_h.at[img], x2_v)
        pltpu.sync_copy(y2_h.at[img], y2_v)
        pltpu.sync_copy(c_h.at[img], c_v)
        pltpu.sync_copy(d_h.at[img], d_v)

        zeros = jnp.zeros((16,), jnp.float32)

        def zbody(j, carry):
            sl = pl.ds(j * 16, 16)
            bs[sl] = zeros
            bx1[sl] = zeros
            by1[sl] = zeros
            bx2[sl] = zeros
            by2[sl] = zeros
            bc[sl] = zeros
            return carry

        lax.fori_loop(0, _PAD // 16, zbody, 0)

        def body(i, carry):
            sl = pl.ds(i * 16, 16)
            pos = d_v[sl]
            plsc.store_scatter(bs, [pos], s_v[sl])
            plsc.store_scatter(bx1, [pos], x1_v[sl])
            plsc.store_scatter(by1, [pos], y1_v[sl])
            plsc.store_scatter(bx2, [pos], x2_v[sl])
            plsc.store_scatter(by2, [pos], y2_v[sl])
            plsc.store_scatter(bc, [pos], c_v[sl])
            return carry

        lax.fori_loop(0, _NB // 16, body, 0)

        tail = pl.ds(_PAD - 16, 16)
        bs[tail] = zeros
        bx1[tail] = zeros
        by1[tail] = zeros
        bx2[tail] = zeros
        by2[tail] = zeros
        bc[tail] = zeros

        pltpu.sync_copy(bs, os_h.at[img])
        pltpu.sync_copy(bx1, ox1_h.at[img])
        pltpu.sync_copy(by1, oy1_h.at[img])
        pltpu.sync_copy(bx2, ox2_h.at[img])
        pltpu.sync_copy(by2, oy2_h.at[img])
        pltpu.sync_copy(bc, oc_h.at[img])


def _compact(s, x1, y1, x2, y2, c, d):
    out = jax.ShapeDtypeStruct((_N, _PAD), jnp.float32)
    mesh = plsc.VectorSubcoreMesh(core_axis_name="c", subcore_axis_name="s")
    big = pltpu.VMEM((_NB,), jnp.float32)
    bigi = pltpu.VMEM((_NB,), jnp.int32)
    small = pltpu.VMEM((_PAD,), jnp.float32)
    return pl.kernel(
        _compact_body,
        out_type=[out] * 6,
        mesh=mesh,
        scratch_types=[big] * 6 + [bigi] + [small] * 6,
        compiler_params=pltpu.CompilerParams(needs_layout_passes=False),
    )(s, x1, y1, x2, y2, c, d)


def _nms_body(s_ref, x1_ref, y1_ref, x2_ref, y2_ref, c_ref,
              ox1, oy1, ox2, oy2, os_, oc):
    s = s_ref[...]
    x1 = x1_ref[...]
    y1 = y1_ref[...]
    x2 = x2_ref[...]
    y2 = y2_ref[...]
    cid = c_ref[...]
    area = (x2 - x1) * (y2 - y1)
    iota = lax.broadcasted_iota(jnp.int32, (_N, _PAD), 1)
    suppr = s < _CONF

    def ext(onehot, arr):
        return jnp.sum(jnp.where(onehot, arr, 0.0), axis=1, keepdims=True)

    for j in range(_MAXDET):
        masked = jnp.where(suppr, -1.0, s)
        m = jnp.max(masked, axis=1, keepdims=True)
        i_min = jnp.min(jnp.where(masked == m, iota, _PAD), axis=1,
                        keepdims=True)
        onehot = iota == i_min
        valid = m >= 0.0
        bx1 = ext(onehot, x1)
        by1 = ext(onehot, y1)
        bx2 = ext(onehot, x2)
        by2 = ext(onehot, y2)
        bc = ext(onehot, cid)
        barea = ext(onehot, area)
        xx1 = jnp.maximum(bx1, x1)
        yy1 = jnp.maximum(by1, y1)
        xx2 = jnp.minimum(bx2, x2)
        yy2 = jnp.minimum(by2, y2)
        inter = jnp.maximum(xx2 - xx1, 0.0) * jnp.maximum(yy2 - yy1, 0.0)
        iou = inter / (barea + area - inter + 1e-16)
        suppr = suppr | (iou > _NMS_T) | onehot
        col = slice(j, j + 1)
        ox1[:, col] = jnp.where(valid, bx1, 0.0)
        oy1[:, col] = jnp.where(valid, by1, 0.0)
        ox2[:, col] = jnp.where(valid, bx2, 0.0)
        oy2[:, col] = jnp.where(valid, by2, 0.0)
        os_[:, col] = jnp.where(valid, m, 0.0)
        oc[:, col] = jnp.where(valid, bc, 0.0)


def _nms(s, x1, y1, x2, y2, c):
    out = jax.ShapeDtypeStruct((_N, 128), jnp.float32)
    return pl.pallas_call(
        _nms_body,
        out_shape=[out] * 6,
    )(s, x1, y1, x2, y2, c)


def kernel(x):
    xt = x.transpose(0, 2, 3, 1).reshape(_N, _S, _A * (_NCLS + 5))
    t = _extract(xt)
    s, x1, y1, x2, y2, c, d = _select(t)
    cs, cx1, cy1, cx2, cy2, cc = _compact(s, x1, y1, x2, y2, c, d)
    ox1, oy1, ox2, oy2, osc, ocl = _nms(cs, cx1, cy1, cx2, cy2, cc)
    rows = jnp.stack([ox1[:, :_MAXDET], oy1[:, :_MAXDET], ox2[:, :_MAXDET],
                      oy2[:, :_MAXDET], osc[:, :_MAXDET], ocl[:, :_MAXDET]],
                     axis=-1)
    return rows


# speedup vs baseline: 3.4040x; 3.4040x over previous
import functools

import jax
import jax.numpy as jnp
import numpy as np
from jax import lax
from jax.experimental import pallas as pl
from jax.experimental.pallas import tpu as pltpu
from jax.experimental.pallas import tpu_sc as plsc

_N = 16
_A = 3
_H = 64
_W = 64
_NCLS = 80
_S = _H * _W
_NB = _A * _S
_K = 300
_PAD = 384
_MAXDET = 100
_CONF = 0.5
_NMS_T = 0.4
_IMG = 512.0
_AW = (10.0, 16.0, 33.0)
_AH = (13.0, 30.0, 23.0)
_ONE_BITS = np.int32(np.float32(1.0).view(np.int32))


def _extract_body(x_ref, o_ref):
    xb = x_ref[0]
    xq = jnp.transpose(xb, (1, 0))
    zero_row = jnp.zeros((_S,), jnp.float32)
    for a in range(_A):
        base = a * (_NCLS + 5)
        cls = xq[base + 5:base + 5 + _NCLS]
        cmax = jnp.max(cls, axis=0)
        li = lax.broadcasted_iota(jnp.int32, (_NCLS, _S), 0)
        cid = jnp.min(jnp.where(cls == cmax[None, :], li, _NCLS), axis=0)

        r = a * 8
        o_ref[0, r + 0] = xq[base + 0]
        o_ref[0, r + 1] = xq[base + 1]
        o_ref[0, r + 2] = xq[base + 2]
        o_ref[0, r + 3] = xq[base + 3]
        o_ref[0, r + 4] = xq[base + 4]
        o_ref[0, r + 5] = cmax
        o_ref[0, r + 6] = cid.astype(jnp.float32)
        o_ref[0, r + 7] = zero_row


def _extract(xt):
    return pl.pallas_call(
        _extract_body,
        grid=(_N,),
        in_specs=[pl.BlockSpec((1, _S, _A * (_NCLS + 5)), lambda n: (n, 0, 0))],
        out_specs=pl.BlockSpec((1, 24, _S), lambda n: (n, 0, 0)),
        out_shape=jax.ShapeDtypeStruct((_N, 24, _S), jnp.float32),
    )(xt)


def _select_body(t_ref, s_ref, x1_ref, y1_ref, x2_ref, y2_ref, c_ref, d_ref):
    t = t_ref[...]

    def field(j):
        return jnp.concatenate([t[:, j, :], t[:, 8 + j, :], t[:, 16 + j, :]],
                               axis=1)

    tx = field(0)
    ty = field(1)
    tw = field(2)
    th = field(3)
    tobj = field(4)
    cmax = field(5)
    cid = field(6)

    lin = lax.broadcasted_iota(jnp.int32, (_N, _NB), 1)
    gx = (lin % _W).astype(jnp.float32)
    gy = ((lin % _S) // _W).astype(jnp.float32)
    aw = jnp.where(lin < _S, _AW[0], jnp.where(lin < 2 * _S, _AW[1], _AW[2]))
    ah = jnp.where(lin < _S, _AH[0], jnp.where(lin < 2 * _S, _AH[1], _AH[2]))

    px = jax.nn.sigmoid(tx) + gx
    py = jax.nn.sigmoid(ty) + gy
    pw = jnp.exp(tw) * aw
    ph = jnp.exp(th) * ah
    bx = (px * _IMG) * _IMG
    by = (py * _IMG) * _IMG
    bw = (pw * _IMG) * _IMG
    bh = (ph * _IMG) * _IMG
    score = jax.nn.sigmoid(tobj) * jax.nn.sigmoid(cmax)

    s_ref[...] = score
    x1_ref[...] = bx - bw / 2.0
    y1_ref[...] = by - bh / 2.0
    x2_ref[...] = bx + bw / 2.0
    y2_ref[...] = by + bh / 2.0
    c_ref[...] = cid

    bits = lax.bitcast_convert_type(score, jnp.int32)

    def b1(_, carry):
        lo, hi = carry
        mid = (lo + hi) // 2
        cnt = jnp.sum((bits >= mid).astype(jnp.int32), axis=1, keepdims=True)
        ok = cnt >= _K
        return jnp.where(ok, mid, lo), jnp.where(ok, hi, mid)

    lo0 = jnp.zeros((_N, 1), jnp.int32)
    hi0 = jnp.full((_N, 1), _ONE_BITS + np.int32(1), jnp.int32)
    tbits, _ = lax.fori_loop(0, 31, b1, (lo0, hi0))
    cnt_gt = jnp.sum((bits >= tbits + 1).astype(jnp.int32), axis=1,
                     keepdims=True)
    need = _K - cnt_gt
    at_t = bits == tbits

    def b2(_, carry):
        lo2, hi2 = carry
        mid = (lo2 + hi2) // 2
        cnt = jnp.sum((at_t & (lin <= mid)).astype(jnp.int32), axis=1,
                      keepdims=True)
        ok = cnt >= need
        return jnp.where(ok, lo2, mid), jnp.where(ok, mid, hi2)

    lo20 = jnp.full((_N, 1), -1, jnp.int32)
    hi20 = jnp.full((_N, 1), _NB - 1, jnp.int32)
    _, cstar = lax.fori_loop(0, 15, b2, (lo20, hi20))
    elig = (bits > tbits) | (at_t & (lin <= cstar))

    e_i = elig.astype(jnp.int32)
    c = e_i
    k = 1
    while k < _NB:
        c = c + jnp.concatenate(
            [jnp.zeros((_N, k), jnp.int32), c[:, :-k]], axis=1)
        k *= 2
    pexcl = c - e_i
    d_ref[...] = jnp.where(elig, pexcl, (_PAD - 16) + (lin & 15))


def _select(t):
    f = jax.ShapeDtypeStruct((_N, _NB), jnp.float32)
    fi = jax.ShapeDtypeStruct((_N, _NB), jnp.int32)
    return pl.pallas_call(
        _select_body,
        out_shape=[f] * 6 + [fi],
    )(t)


def _compact_body(s_h, x1_h, y1_h, x2_h, y2_h, c_h, d_h,
                  os_h, ox1_h, oy1_h, ox2_h, oy2_h, oc_h,
                  s_v, x1_v, y1_v, x2_v, y2_v, c_v, d_v,
                  bs, bx1, by1, bx2, by2, bc):
    wid = lax.axis_index("s") * 2 + lax.axis_index("c")

    @pl.when(wid < _N)
    def _():
        img = wid
        pltpu.sync_copy(s_h.at[img], s_v)
        pltpu.sync_copy(x1_h.at[img], x1_v)
        pltpu.sync_copy(y1_h.at[img], y1_v)
        pltpu.sync_copy(x2_h.at[img], x2_v)
        pltpu.sync_copy(y2_h.at[img], y2_v)
        pltpu.sync_copy(c_h.at[img], c_v)
        pltpu.sync_copy(d_h.at[img], d_v)

        zeros = jnp.zeros((16,), jnp.float32)

        def zbody(j, carry):
            sl = pl.ds(j * 16, 16)
            bs[sl] = zeros
            bx1[sl] = zeros
            by1[sl] = zeros
            bx2[sl] = zeros
            by2[sl] = zeros
            bc[sl] = zeros
            return carry

        lax.fori_loop(0, _PAD // 16, zbody, 0)

        def body(i, carry):
            sl = pl.ds(i * 16, 16)
            pos = d_v[sl]
            plsc.store_scatter(bs, [pos], s_v[sl])
            plsc.store_scatter(bx1, [pos], x1_v[sl])
            plsc.store_scatter(by1, [pos], y1_v[sl])
            plsc.store_scatter(bx2, [pos], x2_v[sl])
            plsc.store_scatter(by2, [pos], y2_v[sl])
            plsc.store_scatter(bc, [pos], c_v[sl])
            return carry

        lax.fori_loop(0, _NB // 16, body, 0)

        tail = pl.ds(_PAD - 16, 16)
        bs[tail] = zeros
        bx1[tail] = zeros
        by1[tail] = zeros
        bx2[tail] = zeros
        by2[tail] = zeros
        bc[tail] = zeros

        pltpu.sync_copy(bs, os_h.at[img])
        pltpu.sync_copy(bx1, ox1_h.at[img])
        pltpu.sync_copy(by1, oy1_h.at[img])
        pltpu.sync_copy(bx2, ox2_h.at[img])
        pltpu.sync_copy(by2, oy2_h.at[img])
        pltpu.sync_copy(bc, oc_h.at[img])


def _compact(s, x1, y1, x2, y2, c, d):
    out = jax.ShapeDtypeStruct((_N, _PAD), jnp.float32)
    mesh = plsc.VectorSubcoreMesh(core_axis_name="c", subcore_axis_name="s")
    big = pltpu.VMEM((_NB,), jnp.float32)
    bigi = pltpu.VMEM((_NB,), jnp.int32)
    small = pltpu.VMEM((_PAD,), jnp.float32)
    return pl.kernel(
        _compact_body,
        out_type=[out] * 6,
        mesh=mesh,
        scratch_types=[big] * 6 + [bigi] + [small] * 6,
        compiler_params=pltpu.CompilerParams(needs_layout_passes=False),
    )(s, x1, y1, x2, y2, c, d)


def _nms_body(s_ref, x1_ref, y1_ref, x2_ref, y2_ref, c_ref,
              ox1, oy1, ox2, oy2, os_, oc):
    s = s_ref[...]
    x1 = x1_ref[...]
    y1 = y1_ref[...]
    x2 = x2_ref[...]
    y2 = y2_ref[...]
    cid = c_ref[...]
    area = (x2 - x1) * (y2 - y1)
    iota = lax.broadcasted_iota(jnp.int32, (_N, _PAD), 1)
    suppr = s < _CONF

    def ext(onehot, arr):
        return jnp.sum(jnp.where(onehot, arr, 0.0), axis=1, keepdims=True)

    for j in range(_MAXDET):
        masked = jnp.where(suppr, -1.0, s)
        m = jnp.max(masked, axis=1, keepdims=True)
        i_min = jnp.min(jnp.where(masked == m, iota, _PAD), axis=1,
                        keepdims=True)
        onehot = iota == i_min
        valid = m >= 0.0
        bx1 = ext(onehot, x1)
        by1 = ext(onehot, y1)
        bx2 = ext(onehot, x2)
        by2 = ext(onehot, y2)
        bc = ext(onehot, cid)
        barea = ext(onehot, area)
        xx1 = jnp.maximum(bx1, x1)
        yy1 = jnp.maximum(by1, y1)
        xx2 = jnp.minimum(bx2, x2)
        yy2 = jnp.minimum(by2, y2)
        inter = jnp.maximum(xx2 - xx1, 0.0) * jnp.maximum(yy2 - yy1, 0.0)
        iou = inter / (barea + area - inter + 1e-16)
        suppr = suppr | (iou > _NMS_T) | onehot
        col = slice(j, j + 1)
        ox1[:, col] = jnp.where(valid, bx1, 0.0)
        oy1[:, col] = jnp.where(valid, by1, 0.0)
        ox2[:, col] = jnp.where(valid, bx2, 0.0)
        oy2[:, col] = jnp.where(valid, by2, 0.0)
        os_[:, col] = jnp.where(valid, m, 0.0)
        oc[:, col] = jnp.where(valid, bc, 0.0)


def _nms(s, x1, y1, x2, y2, c):
    out = jax.ShapeDtypeStruct((_N, 128), jnp.float32)
    return pl.pallas_call(
        _nms_body,
        out_shape=[out] * 6,
    )(s, x1, y1, x2, y2, c)


def kernel(x):
    xt = x.transpose(0, 2, 3, 1).reshape(_N, _S, _A * (_NCLS + 5))
    t = _extract(xt)
    s, x1, y1, x2, y2, c, d = _select(t)
    cs, cx1, cy1, cx2, cy2, cc = _compact(s, x1, y1, x2, y2, c, d)
    ox1, oy1, ox2, oy2, osc, ocl = _nms(cs, cx1, cy1, cx2, cy2, cc)
    rows = jnp.stack([ox1[:, :_MAXDET], oy1[:, :_MAXDET], ox2[:, :_MAXDET],
                      oy2[:, :_MAXDET], osc[:, :_MAXDET], ocl[:, :_MAXDET]],
                     axis=-1)
    return rows


# speedup vs baseline: 3.7616x; 1.1050x over previous
import functools

import jax
import jax.numpy as jnp
import numpy as np
from jax import lax
from jax.experimental import pallas as pl
from jax.experimental.pallas import tpu as pltpu
from jax.experimental.pallas import tpu_sc as plsc

_N = 16
_A = 3
_H = 64
_W = 64
_NCLS = 80
_S = _H * _W
_NB = _A * _S
_K = 300
_PAD = 384
_MAXDET = 100
_CONF = 0.5
_NMS_T = 0.4
_IMG = 512.0
_AW = (10.0, 16.0, 33.0)
_AH = (13.0, 30.0, 23.0)
_ONE_BITS = np.int32(np.float32(1.0).view(np.int32))


def _extract_body(x_ref, o_ref):
    xb = x_ref[0]
    xq = jnp.transpose(xb, (1, 0))
    zero_row = jnp.zeros((_S,), jnp.float32)
    for a in range(_A):
        base = a * (_NCLS + 5)
        cls = xq[base + 5:base + 5 + _NCLS]
        cmax = jnp.max(cls, axis=0)
        li = lax.broadcasted_iota(jnp.int32, (_NCLS, _S), 0)
        cid = jnp.min(jnp.where(cls == cmax[None, :], li, _NCLS), axis=0)

        r = a * 8
        o_ref[0, r + 0] = xq[base + 0]
        o_ref[0, r + 1] = xq[base + 1]
        o_ref[0, r + 2] = xq[base + 2]
        o_ref[0, r + 3] = xq[base + 3]
        o_ref[0, r + 4] = xq[base + 4]
        o_ref[0, r + 5] = cmax
        o_ref[0, r + 6] = cid.astype(jnp.float32)
        o_ref[0, r + 7] = zero_row


def _extract(xt):
    return pl.pallas_call(
        _extract_body,
        grid=(_N,),
        in_specs=[pl.BlockSpec((1, _S, _A * (_NCLS + 5)), lambda n: (n, 0, 0))],
        out_specs=pl.BlockSpec((1, 24, _S), lambda n: (n, 0, 0)),
        out_shape=jax.ShapeDtypeStruct((_N, 24, _S), jnp.float32),
    )(xt)


def _select_body(t_ref, s_ref, x1_ref, y1_ref, x2_ref, y2_ref, c_ref, d_ref):
    t = t_ref[...]

    def field(j):
        return jnp.concatenate([t[:, j, :], t[:, 8 + j, :], t[:, 16 + j, :]],
                               axis=1)

    tx = field(0)
    ty = field(1)
    tw = field(2)
    th = field(3)
    tobj = field(4)
    cmax = field(5)
    cid = field(6)

    lin = lax.broadcasted_iota(jnp.int32, (_N, _NB), 1)
    gx = (lin % _W).astype(jnp.float32)
    gy = ((lin % _S) // _W).astype(jnp.float32)
    aw = jnp.where(lin < _S, _AW[0], jnp.where(lin < 2 * _S, _AW[1], _AW[2]))
    ah = jnp.where(lin < _S, _AH[0], jnp.where(lin < 2 * _S, _AH[1], _AH[2]))

    px = jax.nn.sigmoid(tx) + gx
    py = jax.nn.sigmoid(ty) + gy
    pw = jnp.exp(tw) * aw
    ph = jnp.exp(th) * ah
    bx = (px * _IMG) * _IMG
    by = (py * _IMG) * _IMG
    bw = (pw * _IMG) * _IMG
    bh = (ph * _IMG) * _IMG
    score = jax.nn.sigmoid(tobj) * jax.nn.sigmoid(cmax)

    s_ref[...] = score
    x1_ref[...] = bx - bw / 2.0
    y1_ref[...] = by - bh / 2.0
    x2_ref[...] = bx + bw / 2.0
    y2_ref[...] = by + bh / 2.0
    c_ref[...] = cid

    bits = lax.bitcast_convert_type(score, jnp.int32)

    def b1(_, carry):
        lo, hi = carry
        mid = (lo + hi) // 2
        cnt = jnp.sum((bits >= mid).astype(jnp.int32), axis=1, keepdims=True)
        ok = cnt >= _K
        return jnp.where(ok, mid, lo), jnp.where(ok, hi, mid)

    lo0 = jnp.zeros((_N, 1), jnp.int32)
    hi0 = jnp.full((_N, 1), _ONE_BITS + np.int32(1), jnp.int32)
    tbits, _ = lax.fori_loop(0, 31, b1, (lo0, hi0))
    cnt_gt = jnp.sum((bits >= tbits + 1).astype(jnp.int32), axis=1,
                     keepdims=True)
    need = _K - cnt_gt
    at_t = bits == tbits

    def b2(_, carry):
        lo2, hi2 = carry
        mid = (lo2 + hi2) // 2
        cnt = jnp.sum((at_t & (lin <= mid)).astype(jnp.int32), axis=1,
                      keepdims=True)
        ok = cnt >= need
        return jnp.where(ok, lo2, mid), jnp.where(ok, mid, hi2)

    lo20 = jnp.full((_N, 1), -1, jnp.int32)
    hi20 = jnp.full((_N, 1), _NB - 1, jnp.int32)
    _, cstar = lax.fori_loop(0, 15, b2, (lo20, hi20))
    elig = (bits > tbits) | (at_t & (lin <= cstar))

    e_i = elig.astype(jnp.int32)
    c = e_i
    k = 1
    while k < _NB:
        c = c + jnp.concatenate(
            [jnp.zeros((_N, k), jnp.int32), c[:, :-k]], axis=1)
        k *= 2
    pexcl = c - e_i
    d_ref[...] = jnp.where(elig, pexcl, (_PAD - 16) + (lin & 15))


def _select(t):
    f = jax.ShapeDtypeStruct((_N, _NB), jnp.float32)
    fi = jax.ShapeDtypeStruct((_N, _NB), jnp.int32)
    return pl.pallas_call(
        _select_body,
        out_shape=[f] * 6 + [fi],
    )(t)


def _compact_body(s_h, x1_h, y1_h, x2_h, y2_h, c_h, d_h,
                  os_h, ox1_h, oy1_h, ox2_h, oy2_h, oc_h,
                  s_v, x1_v, y1_v, x2_v, y2_v, c_v, d_v,
                  srcb, bs, bx1, by1, bx2, by2, bc):
    wid = lax.axis_index("s") * 2 + lax.axis_index("c")

    @pl.when(wid < _N)
    def _():
        img = wid
        pltpu.sync_copy(s_h.at[img], s_v)
        pltpu.sync_copy(x1_h.at[img], x1_v)
        pltpu.sync_copy(y1_h.at[img], y1_v)
        pltpu.sync_copy(x2_h.at[img], x2_v)
        pltpu.sync_copy(y2_h.at[img], y2_v)
        pltpu.sync_copy(c_h.at[img], c_v)
        pltpu.sync_copy(d_h.at[img], d_v)

        zeros = jnp.zeros((16,), jnp.float32)
        zeros_i = jnp.zeros((16,), jnp.int32)
        lane = lax.iota(jnp.int32, 16)

        def zbody(j, carry):
            srcb[pl.ds(j * 16, 16)] = zeros_i
            return carry

        lax.fori_loop(0, _PAD // 16, zbody, 0)

        def body(i, carry):
            pos = d_v[pl.ds(i * 16, 16)]
            plsc.store_scatter(srcb, [pos], lane + i * 16)
            return carry

        lax.fori_loop(0, _NB // 16, body, 0)

        def gbody(j, carry):
            sl = pl.ds(j * 16, 16)
            src = srcb[sl]
            bs[sl] = plsc.load_gather(s_v, [src])
            bx1[sl] = plsc.load_gather(x1_v, [src])
            by1[sl] = plsc.load_gather(y1_v, [src])
            bx2[sl] = plsc.load_gather(x2_v, [src])
            by2[sl] = plsc.load_gather(y2_v, [src])
            bc[sl] = plsc.load_gather(c_v, [src])
            return carry

        lax.fori_loop(0, _PAD // 16, gbody, 0)

        kv = pl.ds((_K // 16) * 16, 16)
        keep = (lane + (_K // 16) * 16) < _K
        bs[kv] = jnp.where(keep, bs[kv], 0.0)
        bx1[kv] = jnp.where(keep, bx1[kv], 0.0)
        by1[kv] = jnp.where(keep, by1[kv], 0.0)
        bx2[kv] = jnp.where(keep, bx2[kv], 0.0)
        by2[kv] = jnp.where(keep, by2[kv], 0.0)
        bc[kv] = jnp.where(keep, bc[kv], 0.0)

        def tbody(j, carry):
            sl = pl.ds(j * 16, 16)
            bs[sl] = zeros
            bx1[sl] = zeros
            by1[sl] = zeros
            bx2[sl] = zeros
            by2[sl] = zeros
            bc[sl] = zeros
            return carry

        lax.fori_loop(_K // 16 + 1, _PAD // 16, tbody, 0)

        pltpu.sync_copy(bs, os_h.at[img])
        pltpu.sync_copy(bx1, ox1_h.at[img])
        pltpu.sync_copy(by1, oy1_h.at[img])
        pltpu.sync_copy(bx2, ox2_h.at[img])
        pltpu.sync_copy(by2, oy2_h.at[img])
        pltpu.sync_copy(bc, oc_h.at[img])


def _compact(s, x1, y1, x2, y2, c, d):
    out = jax.ShapeDtypeStruct((_N, _PAD), jnp.float32)
    mesh = plsc.VectorSubcoreMesh(core_axis_name="c", subcore_axis_name="s")
    big = pltpu.VMEM((_NB,), jnp.float32)
    bigi = pltpu.VMEM((_NB,), jnp.int32)
    smalli = pltpu.VMEM((_PAD,), jnp.int32)
    small = pltpu.VMEM((_PAD,), jnp.float32)
    return pl.kernel(
        _compact_body,
        out_type=[out] * 6,
        mesh=mesh,
        scratch_types=[big] * 6 + [bigi] + [smalli] + [small] * 6,
        compiler_params=pltpu.CompilerParams(needs_layout_passes=False),
    )(s, x1, y1, x2, y2, c, d)


def _nms_body(s_ref, x1_ref, y1_ref, x2_ref, y2_ref, c_ref,
              ox1, oy1, ox2, oy2, os_, oc):
    s = s_ref[...]
    x1 = x1_ref[...]
    y1 = y1_ref[...]
    x2 = x2_ref[...]
    y2 = y2_ref[...]
    cid = c_ref[...]
    area = (x2 - x1) * (y2 - y1)
    iota = lax.broadcasted_iota(jnp.int32, (_N, _PAD), 1)
    suppr = s < _CONF

    def ext(onehot, arr):
        return jnp.sum(jnp.where(onehot, arr, 0.0), axis=1, keepdims=True)

    for j in range(_MAXDET):
        masked = jnp.where(suppr, -1.0, s)
        m = jnp.max(masked, axis=1, keepdims=True)
        i_min = jnp.min(jnp.where(masked == m, iota, _PAD), axis=1,
                        keepdims=True)
        onehot = iota == i_min
        valid = m >= 0.0
        bx1 = ext(onehot, x1)
        by1 = ext(onehot, y1)
        bx2 = ext(onehot, x2)
        by2 = ext(onehot, y2)
        bc = ext(onehot, cid)
        barea = ext(onehot, area)
        xx1 = jnp.maximum(bx1, x1)
        yy1 = jnp.maximum(by1, y1)
        xx2 = jnp.minimum(bx2, x2)
        yy2 = jnp.minimum(by2, y2)
        inter = jnp.maximum(xx2 - xx1, 0.0) * jnp.maximum(yy2 - yy1, 0.0)
        iou = inter / (barea + area - inter + 1e-16)
        suppr = suppr | (iou > _NMS_T) | onehot
        col = slice(j, j + 1)
        ox1[:, col] = jnp.where(valid, bx1, 0.0)
        oy1[:, col] = jnp.where(valid, by1, 0.0)
        ox2[:, col] = jnp.where(valid, bx2, 0.0)
        oy2[:, col] = jnp.where(valid, by2, 0.0)
        os_[:, col] = jnp.where(valid, m, 0.0)
        oc[:, col] = jnp.where(valid, bc, 0.0)


def _nms(s, x1, y1, x2, y2, c):
    out = jax.ShapeDtypeStruct((_N, 128), jnp.float32)
    return pl.pallas_call(
        _nms_body,
        out_shape=[out] * 6,
    )(s, x1, y1, x2, y2, c)


def kernel(x):
    xt = x.transpose(0, 2, 3, 1).reshape(_N, _S, _A * (_NCLS + 5))
    t = _extract(xt)
    s, x1, y1, x2, y2, c, d = _select(t)
    cs, cx1, cy1, cx2, cy2, cc = _compact(s, x1, y1, x2, y2, c, d)
    ox1, oy1, ox2, oy2, osc, ocl = _nms(cs, cx1, cy1, cx2, cy2, cc)
    rows = jnp.stack([ox1[:, :_MAXDET], oy1[:, :_MAXDET], ox2[:, :_MAXDET],
                      oy2[:, :_MAXDET], osc[:, :_MAXDET], ocl[:, :_MAXDET]],
                     axis=-1)
    return rows


# speedup vs baseline: 3.7655x; 1.0010x over previous
import functools

import jax
import jax.numpy as jnp
import numpy as np
from jax import lax
from jax.experimental import pallas as pl
from jax.experimental.pallas import tpu as pltpu
from jax.experimental.pallas import tpu_sc as plsc

_N = 16
_A = 3
_H = 64
_W = 64
_NCLS = 80
_S = _H * _W
_NB = _A * _S
_K = 300
_PAD = 384
_MAXDET = 100
_CONF = 0.5
_NMS_T = 0.4
_IMG = 512.0
_AW = (10.0, 16.0, 33.0)
_AH = (13.0, 30.0, 23.0)
_ONE_BITS = np.int32(np.float32(1.0).view(np.int32))


def _extract_body(x_ref, o_ref):
    xb = x_ref[0]
    xq = jnp.transpose(xb, (1, 0))
    zero_row = jnp.zeros((_S,), jnp.float32)
    for a in range(_A):
        base = a * (_NCLS + 5)
        cls = xq[base + 5:base + 5 + _NCLS]
        cmax = jnp.max(cls, axis=0)
        li = lax.broadcasted_iota(jnp.int32, (_NCLS, _S), 0)
        cid = jnp.min(jnp.where(cls == cmax[None, :], li, _NCLS), axis=0)

        r = a * 8
        o_ref[0, r + 0] = xq[base + 0]
        o_ref[0, r + 1] = xq[base + 1]
        o_ref[0, r + 2] = xq[base + 2]
        o_ref[0, r + 3] = xq[base + 3]
        o_ref[0, r + 4] = xq[base + 4]
        o_ref[0, r + 5] = cmax
        o_ref[0, r + 6] = cid.astype(jnp.float32)
        o_ref[0, r + 7] = zero_row


def _extract(xt):
    return pl.pallas_call(
        _extract_body,
        grid=(_N,),
        in_specs=[pl.BlockSpec((1, _S, _A * (_NCLS + 5)), lambda n: (n, 0, 0))],
        out_specs=pl.BlockSpec((1, 24, _S), lambda n: (n, 0, 0)),
        out_shape=jax.ShapeDtypeStruct((_N, 24, _S), jnp.float32),
    )(xt)


def _select_body(t_ref, s_ref, x1_ref, y1_ref, x2_ref, y2_ref, c_ref, d_ref):
    t = t_ref[...]

    def field(j):
        return jnp.concatenate([t[:, j, :], t[:, 8 + j, :], t[:, 16 + j, :]],
                               axis=1)

    tx = field(0)
    ty = field(1)
    tw = field(2)
    th = field(3)
    tobj = field(4)
    cmax = field(5)
    cid = field(6)

    lin = lax.broadcasted_iota(jnp.int32, (_N, _NB), 1)
    gx = (lin % _W).astype(jnp.float32)
    gy = ((lin % _S) // _W).astype(jnp.float32)
    aw = jnp.where(lin < _S, _AW[0], jnp.where(lin < 2 * _S, _AW[1], _AW[2]))
    ah = jnp.where(lin < _S, _AH[0], jnp.where(lin < 2 * _S, _AH[1], _AH[2]))

    px = jax.nn.sigmoid(tx) + gx
    py = jax.nn.sigmoid(ty) + gy
    pw = jnp.exp(tw) * aw
    ph = jnp.exp(th) * ah
    bx = (px * _IMG) * _IMG
    by = (py * _IMG) * _IMG
    bw = (pw * _IMG) * _IMG
    bh = (ph * _IMG) * _IMG
    score = jax.nn.sigmoid(tobj) * jax.nn.sigmoid(cmax)

    s_ref[...] = score
    x1_ref[...] = bx - bw / 2.0
    y1_ref[...] = by - bh / 2.0
    x2_ref[...] = bx + bw / 2.0
    y2_ref[...] = by + bh / 2.0
    c_ref[...] = cid

    bits = lax.bitcast_convert_type(score, jnp.int32)

    def b1(_, carry):
        lo, hi = carry
        mid = (lo + hi) // 2
        cnt = jnp.sum((bits >= mid).astype(jnp.int32), axis=1, keepdims=True)
        ok = cnt >= _K
        return jnp.where(ok, mid, lo), jnp.where(ok, hi, mid)

    lo0 = jnp.zeros((_N, 1), jnp.int32)
    hi0 = jnp.full((_N, 1), _ONE_BITS + np.int32(1), jnp.int32)
    tbits, _ = lax.fori_loop(0, 31, b1, (lo0, hi0))
    cnt_gt = jnp.sum((bits >= tbits + 1).astype(jnp.int32), axis=1,
                     keepdims=True)
    need = _K - cnt_gt
    at_t = bits == tbits

    def b2(_, carry):
        lo2, hi2 = carry
        mid = (lo2 + hi2) // 2
        cnt = jnp.sum((at_t & (lin <= mid)).astype(jnp.int32), axis=1,
                      keepdims=True)
        ok = cnt >= need
        return jnp.where(ok, lo2, mid), jnp.where(ok, mid, hi2)

    lo20 = jnp.full((_N, 1), -1, jnp.int32)
    hi20 = jnp.full((_N, 1), _NB - 1, jnp.int32)
    _, cstar = lax.fori_loop(0, 15, b2, (lo20, hi20))
    elig = (bits > tbits) | (at_t & (lin <= cstar))

    e_i = elig.astype(jnp.int32)
    c = e_i
    k = 1
    while k < _NB:
        c = c + jnp.concatenate(
            [jnp.zeros((_N, k), jnp.int32), c[:, :-k]], axis=1)
        k *= 2
    pexcl = c - e_i
    d_ref[...] = jnp.where(elig, pexcl, (_PAD - 16) + (lin & 15))


def _select(t):
    f = jax.ShapeDtypeStruct((_N, _NB), jnp.float32)
    fi = jax.ShapeDtypeStruct((_N, _NB), jnp.int32)
    return pl.pallas_call(
        _select_body,
        out_shape=[f] * 6 + [fi],
    )(t)


def _compact_body(s_h, x1_h, y1_h, x2_h, y2_h, c_h, d_h,
                  os_h, ox1_h, oy1_h, ox2_h, oy2_h, oc_h,
                  s_v, x1_v, y1_v, x2_v, y2_v, c_v, d_v,
                  srcb, bs, bx1, by1, bx2, by2, bc):
    wid = lax.axis_index("s") * 2 + lax.axis_index("c")

    @pl.when(wid < _N)
    def _():
        img = wid
        pltpu.sync_copy(s_h.at[img], s_v)
        pltpu.sync_copy(x1_h.at[img], x1_v)
        pltpu.sync_copy(y1_h.at[img], y1_v)
        pltpu.sync_copy(x2_h.at[img], x2_v)
        pltpu.sync_copy(y2_h.at[img], y2_v)
        pltpu.sync_copy(c_h.at[img], c_v)
        pltpu.sync_copy(d_h.at[img], d_v)

        zeros = jnp.zeros((16,), jnp.float32)
        zeros_i = jnp.zeros((16,), jnp.int32)
        lane = lax.iota(jnp.int32, 16)

        def zbody(j, carry):
            srcb[pl.ds(j * 16, 16)] = zeros_i
            return carry

        lax.fori_loop(0, _PAD // 16, zbody, 0)

        def body(i, carry):
            pos = d_v[pl.ds(i * 16, 16)]
            plsc.store_scatter(srcb, [pos], lane + i * 16)
            return carry

        lax.fori_loop(0, _NB // 16, body, 0)

        def gbody(j, carry):
            sl = pl.ds(j * 16, 16)
            src = srcb[sl]
            bs[sl] = plsc.load_gather(s_v, [src])
            bx1[sl] = plsc.load_gather(x1_v, [src])
            by1[sl] = plsc.load_gather(y1_v, [src])
            bx2[sl] = plsc.load_gather(x2_v, [src])
            by2[sl] = plsc.load_gather(y2_v, [src])
            bc[sl] = plsc.load_gather(c_v, [src])
            return carry

        lax.fori_loop(0, _PAD // 16, gbody, 0)

        kv = pl.ds((_K // 16) * 16, 16)
        keep = (lane + (_K // 16) * 16) < _K
        bs[kv] = jnp.where(keep, bs[kv], 0.0)
        bx1[kv] = jnp.where(keep, bx1[kv], 0.0)
        by1[kv] = jnp.where(keep, by1[kv], 0.0)
        bx2[kv] = jnp.where(keep, bx2[kv], 0.0)
        by2[kv] = jnp.where(keep, by2[kv], 0.0)
        bc[kv] = jnp.where(keep, bc[kv], 0.0)

        def tbody(j, carry):
            sl = pl.ds(j * 16, 16)
            bs[sl] = zeros
            bx1[sl] = zeros
            by1[sl] = zeros
            bx2[sl] = zeros
            by2[sl] = zeros
            bc[sl] = zeros
            return carry

        lax.fori_loop(_K // 16 + 1, _PAD // 16, tbody, 0)

        pltpu.sync_copy(bs, os_h.at[img])
        pltpu.sync_copy(bx1, ox1_h.at[img])
        pltpu.sync_copy(by1, oy1_h.at[img])
        pltpu.sync_copy(bx2, ox2_h.at[img])
        pltpu.sync_copy(by2, oy2_h.at[img])
        pltpu.sync_copy(bc, oc_h.at[img])


def _compact(s, x1, y1, x2, y2, c, d):
    out = jax.ShapeDtypeStruct((_N, _PAD), jnp.float32)
    mesh = plsc.VectorSubcoreMesh(core_axis_name="c", subcore_axis_name="s")
    big = pltpu.VMEM((_NB,), jnp.float32)
    bigi = pltpu.VMEM((_NB,), jnp.int32)
    smalli = pltpu.VMEM((_PAD,), jnp.int32)
    small = pltpu.VMEM((_PAD,), jnp.float32)
    return pl.kernel(
        _compact_body,
        out_type=[out] * 6,
        mesh=mesh,
        scratch_types=[big] * 6 + [bigi] + [smalli] + [small] * 6,
        compiler_params=pltpu.CompilerParams(needs_layout_passes=False),
    )(s, x1, y1, x2, y2, c, d)


def _nms_body(s_ref, x1_ref, y1_ref, x2_ref, y2_ref, c_ref,
              ox1, oy1, ox2, oy2, os_, oc):
    s = s_ref[...]
    x1 = x1_ref[...]
    y1 = y1_ref[...]
    x2 = x2_ref[...]
    y2 = y2_ref[...]
    cid = c_ref[...]
    area = (x2 - x1) * (y2 - y1)
    iota = lax.broadcasted_iota(jnp.int32, (_N, _PAD), 1)
    suppr = s < _CONF
    stack = jnp.concatenate([x1, y1, x2, y2, cid, area], axis=0)

    for j in range(_MAXDET):
        masked = jnp.where(suppr, -1.0, s)
        m = jnp.max(masked, axis=1, keepdims=True)
        i_min = jnp.min(jnp.where(masked == m, iota, _PAD), axis=1,
                        keepdims=True)
        onehot = iota == i_min
        valid = m >= 0.0
        oh6 = jnp.concatenate([onehot] * 6, axis=0)
        ext_all = jnp.sum(jnp.where(oh6, stack, 0.0), axis=1, keepdims=True)
        bx1 = ext_all[0:_N]
        by1 = ext_all[_N:2 * _N]
        bx2 = ext_all[2 * _N:3 * _N]
        by2 = ext_all[3 * _N:4 * _N]
        bc = ext_all[4 * _N:5 * _N]
        barea = ext_all[5 * _N:6 * _N]
        xx1 = jnp.maximum(bx1, x1)
        yy1 = jnp.maximum(by1, y1)
        xx2 = jnp.minimum(bx2, x2)
        yy2 = jnp.minimum(by2, y2)
        inter = jnp.maximum(xx2 - xx1, 0.0) * jnp.maximum(yy2 - yy1, 0.0)
        iou = inter / (barea + area - inter + 1e-16)
        suppr = suppr | (iou > _NMS_T) | onehot
        col = slice(j, j + 1)
        ox1[:, col] = jnp.where(valid, bx1, 0.0)
        oy1[:, col] = jnp.where(valid, by1, 0.0)
        ox2[:, col] = jnp.where(valid, bx2, 0.0)
        oy2[:, col] = jnp.where(valid, by2, 0.0)
        os_[:, col] = jnp.where(valid, m, 0.0)
        oc[:, col] = jnp.where(valid, bc, 0.0)


def _nms(s, x1, y1, x2, y2, c):
    out = jax.ShapeDtypeStruct((_N, 128), jnp.float32)
    return pl.pallas_call(
        _nms_body,
        out_shape=[out] * 6,
    )(s, x1, y1, x2, y2, c)


def kernel(x):
    xt = x.transpose(0, 2, 3, 1).reshape(_N, _S, _A * (_NCLS + 5))
    t = _extract(xt)
    s, x1, y1, x2, y2, c, d = _select(t)
    cs, cx1, cy1, cx2, cy2, cc = _compact(s, x1, y1, x2, y2, c, d)
    ox1, oy1, ox2, oy2, osc, ocl = _nms(cs, cx1, cy1, cx2, cy2, cc)
    rows = jnp.stack([ox1[:, :_MAXDET], oy1[:, :_MAXDET], ox2[:, :_MAXDET],
                      oy2[:, :_MAXDET], osc[:, :_MAXDET], ocl[:, :_MAXDET]],
                     axis=-1)
    return rows


# speedup vs baseline: 3.9499x; 1.0490x over previous
import functools

import jax
import jax.numpy as jnp
import numpy as np
from jax import lax
from jax.experimental import pallas as pl
from jax.experimental.pallas import tpu as pltpu
from jax.experimental.pallas import tpu_sc as plsc

_N = 16
_A = 3
_H = 64
_W = 64
_NCLS = 80
_S = _H * _W
_NB = _A * _S
_K = 300
_PAD = 384
_MAXDET = 100
_CONF = 0.5
_NMS_T = 0.4
_IMG = 512.0
_AW = (10.0, 16.0, 33.0)
_AH = (13.0, 30.0, 23.0)
_ONE_BITS = np.int32(np.float32(1.0).view(np.int32))


def _extract_body(x_ref, o_ref):
    xb = x_ref[0]
    xq = jnp.transpose(xb, (1, 0))
    zero_row = jnp.zeros((_S,), jnp.float32)
    for a in range(_A):
        base = a * (_NCLS + 5)
        cls = xq[base + 5:base + 5 + _NCLS]
        cmax = jnp.max(cls, axis=0)
        li = lax.broadcasted_iota(jnp.int32, (_NCLS, _S), 0)
        cid = jnp.min(jnp.where(cls == cmax[None, :], li, _NCLS), axis=0)

        r = a * 8
        o_ref[0, r + 0] = xq[base + 0]
        o_ref[0, r + 1] = xq[base + 1]
        o_ref[0, r + 2] = xq[base + 2]
        o_ref[0, r + 3] = xq[base + 3]
        o_ref[0, r + 4] = xq[base + 4]
        o_ref[0, r + 5] = cmax
        o_ref[0, r + 6] = cid.astype(jnp.float32)
        o_ref[0, r + 7] = zero_row


def _extract(xt):
    return pl.pallas_call(
        _extract_body,
        grid=(_N,),
        in_specs=[pl.BlockSpec((1, _S, _A * (_NCLS + 5)), lambda n: (n, 0, 0))],
        out_specs=pl.BlockSpec((1, 24, _S), lambda n: (n, 0, 0)),
        out_shape=jax.ShapeDtypeStruct((_N, 24, _S), jnp.float32),
    )(xt)


def _select_body(t_ref, s_ref, x1_ref, y1_ref, x2_ref, y2_ref, c_ref, d_ref):
    t = t_ref[...]

    def field(j):
        return jnp.concatenate([t[:, j, :], t[:, 8 + j, :], t[:, 16 + j, :]],
                               axis=1)

    tx = field(0)
    ty = field(1)
    tw = field(2)
    th = field(3)
    tobj = field(4)
    cmax = field(5)
    cid = field(6)

    lin = lax.broadcasted_iota(jnp.int32, (_N, _NB), 1)
    gx = (lin % _W).astype(jnp.float32)
    gy = ((lin % _S) // _W).astype(jnp.float32)
    aw = jnp.where(lin < _S, _AW[0], jnp.where(lin < 2 * _S, _AW[1], _AW[2]))
    ah = jnp.where(lin < _S, _AH[0], jnp.where(lin < 2 * _S, _AH[1], _AH[2]))

    px = jax.nn.sigmoid(tx) + gx
    py = jax.nn.sigmoid(ty) + gy
    pw = jnp.exp(tw) * aw
    ph = jnp.exp(th) * ah
    bx = (px * _IMG) * _IMG
    by = (py * _IMG) * _IMG
    bw = (pw * _IMG) * _IMG
    bh = (ph * _IMG) * _IMG
    score = jax.nn.sigmoid(tobj) * jax.nn.sigmoid(cmax)

    s_ref[...] = score
    x1_ref[...] = bx - bw / 2.0
    y1_ref[...] = by - bh / 2.0
    x2_ref[...] = bx + bw / 2.0
    y2_ref[...] = by + bh / 2.0
    c_ref[...] = cid

    bits = lax.bitcast_convert_type(score, jnp.int32)

    def b1(_, carry):
        lo, hi = carry
        mid = (lo + hi) // 2
        cnt = jnp.sum((bits >= mid).astype(jnp.int32), axis=1, keepdims=True)
        ok = cnt >= _K
        return jnp.where(ok, mid, lo), jnp.where(ok, hi, mid)

    lo0 = jnp.zeros((_N, 1), jnp.int32)
    hi0 = jnp.full((_N, 1), _ONE_BITS + np.int32(1), jnp.int32)
    tbits, _ = lax.fori_loop(0, 31, b1, (lo0, hi0))
    cnt_gt = jnp.sum((bits >= tbits + 1).astype(jnp.int32), axis=1,
                     keepdims=True)
    need = _K - cnt_gt
    at_t = bits == tbits

    def b2(_, carry):
        lo2, hi2 = carry
        mid = (lo2 + hi2) // 2
        cnt = jnp.sum((at_t & (lin <= mid)).astype(jnp.int32), axis=1,
                      keepdims=True)
        ok = cnt >= need
        return jnp.where(ok, lo2, mid), jnp.where(ok, mid, hi2)

    lo20 = jnp.full((_N, 1), -1, jnp.int32)
    hi20 = jnp.full((_N, 1), _NB - 1, jnp.int32)
    _, cstar = lax.fori_loop(0, 15, b2, (lo20, hi20))
    elig = (bits > tbits) | (at_t & (lin <= cstar))

    e_i = elig.astype(jnp.int32)
    c = e_i
    k = 1
    while k < _NB:
        c = c + jnp.concatenate(
            [jnp.zeros((_N, k), jnp.int32), c[:, :-k]], axis=1)
        k *= 2
    pexcl = c - e_i
    d_ref[...] = jnp.where(elig, pexcl, (_PAD - 16) + (lin & 15))


def _select(t):
    f = jax.ShapeDtypeStruct((_N, _NB), jnp.float32)
    fi = jax.ShapeDtypeStruct((_N, _NB), jnp.int32)
    return pl.pallas_call(
        _select_body,
        out_shape=[f] * 6 + [fi],
    )(t)


def _compact_body(s_h, x1_h, y1_h, x2_h, y2_h, c_h, d_h,
                  os_h, ox1_h, oy1_h, ox2_h, oy2_h, oc_h,
                  s_v, x1_v, y1_v, x2_v, y2_v, c_v, d_v,
                  srcb, bs, bx1, by1, bx2, by2, bc, sem):
    wid = lax.axis_index("s") * 2 + lax.axis_index("c")

    @pl.when(wid < _N)
    def _():
        img = wid
        cps = [pltpu.make_async_copy(h.at[img], v, sem)
               for h, v in ((s_h, s_v), (x1_h, x1_v), (y1_h, y1_v),
                            (x2_h, x2_v), (y2_h, y2_v), (c_h, c_v))]
        for cp in cps:
            cp.start()
        pltpu.sync_copy(d_h.at[img], d_v)

        zeros = jnp.zeros((16,), jnp.float32)
        zeros_i = jnp.zeros((16,), jnp.int32)
        lane = lax.iota(jnp.int32, 16)

        def zbody(j, carry):
            srcb[pl.ds(j * 16, 16)] = zeros_i
            return carry

        lax.fori_loop(0, _PAD // 16, zbody, 0)

        def body(i, carry):
            pos = d_v[pl.ds(i * 16, 16)]
            plsc.store_scatter(srcb, [pos], lane + i * 16)
            return carry

        lax.fori_loop(0, _NB // 16, body, 0)

        for cp in cps:
            cp.wait()

        def gbody(j, carry):
            sl = pl.ds(j * 16, 16)
            src = srcb[sl]
            bs[sl] = plsc.load_gather(s_v, [src])
            bx1[sl] = plsc.load_gather(x1_v, [src])
            by1[sl] = plsc.load_gather(y1_v, [src])
            bx2[sl] = plsc.load_gather(x2_v, [src])
            by2[sl] = plsc.load_gather(y2_v, [src])
            bc[sl] = plsc.load_gather(c_v, [src])
            return carry

        lax.fori_loop(0, _PAD // 16, gbody, 0)

        kv = pl.ds((_K // 16) * 16, 16)
        keep = (lane + (_K // 16) * 16) < _K
        bs[kv] = jnp.where(keep, bs[kv], 0.0)
        bx1[kv] = jnp.where(keep, bx1[kv], 0.0)
        by1[kv] = jnp.where(keep, by1[kv], 0.0)
        bx2[kv] = jnp.where(keep, bx2[kv], 0.0)
        by2[kv] = jnp.where(keep, by2[kv], 0.0)
        bc[kv] = jnp.where(keep, bc[kv], 0.0)

        def tbody(j, carry):
            sl = pl.ds(j * 16, 16)
            bs[sl] = zeros
            bx1[sl] = zeros
            by1[sl] = zeros
            bx2[sl] = zeros
            by2[sl] = zeros
            bc[sl] = zeros
            return carry

        lax.fori_loop(_K // 16 + 1, _PAD // 16, tbody, 0)

        pltpu.sync_copy(bs, os_h.at[img])
        pltpu.sync_copy(bx1, ox1_h.at[img])
        pltpu.sync_copy(by1, oy1_h.at[img])
        pltpu.sync_copy(bx2, ox2_h.at[img])
        pltpu.sync_copy(by2, oy2_h.at[img])
        pltpu.sync_copy(bc, oc_h.at[img])


def _compact(s, x1, y1, x2, y2, c, d):
    out = jax.ShapeDtypeStruct((_N, _PAD), jnp.float32)
    mesh = plsc.VectorSubcoreMesh(core_axis_name="c", subcore_axis_name="s")
    big = pltpu.VMEM((_NB,), jnp.float32)
    bigi = pltpu.VMEM((_NB,), jnp.int32)
    smalli = pltpu.VMEM((_PAD,), jnp.int32)
    small = pltpu.VMEM((_PAD,), jnp.float32)
    return pl.kernel(
        _compact_body,
        out_type=[out] * 6,
        mesh=mesh,
        scratch_types=[big] * 6 + [bigi] + [smalli] + [small] * 6
        + [pltpu.SemaphoreType.DMA],
        compiler_params=pltpu.CompilerParams(needs_layout_passes=False),
    )(s, x1, y1, x2, y2, c, d)


def _nms_body(s_ref, x1_ref, y1_ref, x2_ref, y2_ref, c_ref,
              ox1, oy1, ox2, oy2, os_, oc):
    s = s_ref[...]
    x1 = x1_ref[...]
    y1 = y1_ref[...]
    x2 = x2_ref[...]
    y2 = y2_ref[...]
    cid = c_ref[...]
    area = (x2 - x1) * (y2 - y1)
    iota = lax.broadcasted_iota(jnp.int32, (_N, _PAD), 1)
    suppr = s < _CONF
    stack = jnp.concatenate([x1, y1, x2, y2, cid, area], axis=0)

    for j in range(_MAXDET):
        masked = jnp.where(suppr, -1.0, s)
        m = jnp.max(masked, axis=1, keepdims=True)
        i_min = jnp.min(jnp.where(masked == m, iota, _PAD), axis=1,
                        keepdims=True)
        onehot = iota == i_min
        valid = m >= 0.0
        oh6 = jnp.concatenate([onehot] * 6, axis=0)
        ext_all = jnp.sum(jnp.where(oh6, stack, 0.0), axis=1, keepdims=True)
        bx1 = ext_all[0:_N]
        by1 = ext_all[_N:2 * _N]
        bx2 = ext_all[2 * _N:3 * _N]
        by2 = ext_all[3 * _N:4 * _N]
        bc = ext_all[4 * _N:5 * _N]
        barea = ext_all[5 * _N:6 * _N]
        xx1 = jnp.maximum(bx1, x1)
        yy1 = jnp.maximum(by1, y1)
        xx2 = jnp.minimum(bx2, x2)
        yy2 = jnp.minimum(by2, y2)
        inter = jnp.maximum(xx2 - xx1, 0.0) * jnp.maximum(yy2 - yy1, 0.0)
        iou = inter / (barea + area - inter + 1e-16)
        suppr = suppr | (iou > _NMS_T) | onehot
        col = slice(j, j + 1)
        ox1[:, col] = jnp.where(valid, bx1, 0.0)
        oy1[:, col] = jnp.where(valid, by1, 0.0)
        ox2[:, col] = jnp.where(valid, bx2, 0.0)
        oy2[:, col] = jnp.where(valid, by2, 0.0)
        os_[:, col] = jnp.where(valid, m, 0.0)
        oc[:, col] = jnp.where(valid, bc, 0.0)


def _nms(s, x1, y1, x2, y2, c):
    out = jax.ShapeDtypeStruct((_N, 128), jnp.float32)
    return pl.pallas_call(
        _nms_body,
        out_shape=[out] * 6,
    )(s, x1, y1, x2, y2, c)


def kernel(x):
    xt = x.transpose(0, 2, 3, 1).reshape(_N, _S, _A * (_NCLS + 5))
    t = _extract(xt)
    s, x1, y1, x2, y2, c, d = _select(t)
    cs, cx1, cy1, cx2, cy2, cc = _compact(s, x1, y1, x2, y2, c, d)
    ox1, oy1, ox2, oy2, osc, ocl = _nms(cs, cx1, cy1, cx2, cy2, cc)
    rows = jnp.stack([ox1[:, :_MAXDET], oy1[:, :_MAXDET], ox2[:, :_MAXDET],
                      oy2[:, :_MAXDET], osc[:, :_MAXDET], ocl[:, :_MAXDET]],
                     axis=-1)
    return rows
